# baseline (speedup 1.0000x reference)
import jax
import jax.numpy as jnp
from jax import lax
from jax.experimental import pallas as pl
from jax.experimental.pallas import tpu as pltpu

_DIM = 2048
_K = 1433
_ROWS = 16384
_CHUNK = 1024
_NBLK = _ROWS // _CHUNK
_Q = 4096 // _CHUNK
_RSTEPS = 2 * _Q
_GROUPS = _CHUNK // 8


def _sublane_tree(a):
    b = a[0:4, :] + a[4:8, :]
    c = b[0:2, :] + b[2:4, :]
    return c[0:1, :] + c[1:2, :]


def _build_mask(sumsq):
    imp = jnp.sqrt(sumsq)
    u = lax.bitcast_convert_type(imp, jnp.int32)

    def bs(i, m):
        t = m | lax.shift_left(jnp.int32(1), jnp.int32(30) - i)
        cnt = jnp.sum((u >= t).astype(jnp.int32))
        return jnp.where(cnt >= _K, t, m)

    tau = lax.fori_loop(0, 31, bs, jnp.int32(0))

    gt = u > tau
    eq = u == tau
    n_gt = jnp.sum(gt.astype(jnp.int32))
    need = _K - n_gt
    e = eq.astype(jnp.int32)
    pre = e
    d = 1
    while d < _DIM:
        pre = pre + jnp.concatenate(
            [jnp.zeros((1, d), jnp.int32), pre[:, : _DIM - d]], axis=1
        )
        d *= 2
    excl = pre - e
    keep = jnp.logical_or(gt, jnp.logical_and(eq, excl < need))
    return keep.astype(jnp.float32)


def _fused_body(xa_ref, xb_ref, mask_ref, o_ref, acc_ref, t0_ref):
    i = pl.program_id(0)

    @pl.when(i < _RSTEPS)
    def _():
        h = i // _Q
        s = i % _Q

        @pl.when(s == 0)
        def _():
            acc_ref[...] = jnp.zeros_like(acc_ref)

        def step(g, acc):
            a = xa_ref[pl.ds(g * 8, 8), :]
            acc = acc + a * a
            b = xb_ref[pl.ds(g * 8, 8), :]
            acc = acc + b * b
            return acc

        acc_ref[...] = lax.fori_loop(0, _GROUPS, step, acc_ref[...])

        @pl.when(jnp.logical_and(h == 0, s == _Q - 1))
        def _():
            t0_ref[...] = _sublane_tree(acc_ref[...])

        @pl.when(jnp.logical_and(h == 1, s == _Q - 1))
        def _():
            imp = t0_ref[...] + _sublane_tree(acc_ref[...])
            mask_ref[...] = _build_mask(imp)

    @pl.when(i >= _RSTEPS)
    def _():
        o_ref[...] = xa_ref[...] * mask_ref[...]


def _xa_map(i):
    h = i // _Q
    s = i % _Q
    return (jnp.where(i < _RSTEPS, h * 2 * _Q + s, i - _RSTEPS), 0)


def _xb_map(i):
    h = i // _Q
    s = i % _Q
    return (jnp.where(i < _RSTEPS, h * 2 * _Q + s + _Q, _NBLK - 1), 0)


def kernel(x):
    x2 = x.reshape(_ROWS, _DIM)
    mask, out = pl.pallas_call(
        _fused_body,
        grid=(_RSTEPS + _NBLK,),
        in_specs=[
            pl.BlockSpec((_CHUNK, _DIM), _xa_map),
            pl.BlockSpec((_CHUNK, _DIM), _xb_map),
        ],
        out_specs=[
            pl.BlockSpec((1, _DIM), lambda i: (0, 0)),
            pl.BlockSpec(
                (_CHUNK, _DIM),
                lambda i: (jnp.where(i < _RSTEPS, 0, i - _RSTEPS), 0),
            ),
        ],
        out_shape=[
            jax.ShapeDtypeStruct((1, _DIM), jnp.float32),
            jax.ShapeDtypeStruct((_ROWS, _DIM), jnp.float32),
        ],
        scratch_shapes=[
            pltpu.VMEM((8, _DIM), jnp.float32),
            pltpu.VMEM((1, _DIM), jnp.float32),
        ],
        compiler_params=pltpu.CompilerParams(
            dimension_semantics=("arbitrary",),
            vmem_limit_bytes=63 * 1024 * 1024,
        ),
    )(x2, x2)
    del mask
    return out.reshape(x.shape)
